# Initial kernel scaffold; baseline (speedup 1.0000x reference)
import jax
import jax.numpy as jnp
from jax.experimental import pallas as pl


def kernel(position_ids, table):
    raise NotImplementedError("write your pallas kernel here")


# speedup vs baseline: 1.9823x; 1.9823x over previous
import functools

import jax
import jax.numpy as jnp
from jax import lax
from jax.experimental import pallas as pl
from jax.experimental.pallas import tpu as pltpu
from jax.experimental.pallas import tpu_sc as plsc

MAX_CONTEXT_LENGTH = 8192
D_MODEL = 1024
BATCH = 4
SEQ_LEN = 8192

NTOT = BATCH * SEQ_LEN
NW = 32
B_PER_W = NTOT // NW
CHUNK = 32
NCHUNK = B_PER_W // CHUNK

_mesh = plsc.VectorSubcoreMesh(core_axis_name="c", subcore_axis_name="s")


@functools.partial(
    pl.kernel,
    mesh=_mesh,
    out_type=jax.ShapeDtypeStruct((NTOT, D_MODEL), jnp.float32),
    scratch_types=[
        pltpu.VMEM((B_PER_W,), jnp.int32),
        pltpu.VMEM((CHUNK, D_MODEL), jnp.float32),
        pltpu.SemaphoreType.DMA,
    ],
)
def _gather_kernel(ids_hbm, table_hbm, out_hbm, idx_v, rows_v, sem):
    wid = lax.axis_index("s") * 2 + lax.axis_index("c")
    base = wid * B_PER_W
    pltpu.sync_copy(ids_hbm.at[pl.ds(base, B_PER_W)], idx_v)

    def body(g, carry):
        off = g * CHUNK
        pltpu.async_copy(
            table_hbm.at[idx_v.at[pl.ds(off, CHUNK)]], rows_v, sem
        ).wait()
        pltpu.sync_copy(rows_v, out_hbm.at[pl.ds(base + off, CHUNK)])
        return carry

    lax.fori_loop(0, NCHUNK, body, 0)


def kernel(position_ids, table):
    ids_flat = position_ids.reshape(-1).astype(jnp.int32)
    out = _gather_kernel(ids_flat, table)
    return out.reshape(BATCH, SEQ_LEN, D_MODEL)


# speedup vs baseline: 2.2974x; 1.1590x over previous
import functools

import jax
import jax.numpy as jnp
from jax import lax
from jax.experimental import pallas as pl
from jax.experimental.pallas import tpu as pltpu
from jax.experimental.pallas import tpu_sc as plsc

MAX_CONTEXT_LENGTH = 8192
D_MODEL = 1024
BATCH = 4
SEQ_LEN = 8192

NTOT = BATCH * SEQ_LEN
NW = 32
B_PER_W = NTOT // NW
CHUNK = 32
NCHUNK = B_PER_W // CHUNK
NROUND = NCHUNK // 2

_mesh = plsc.VectorSubcoreMesh(core_axis_name="c", subcore_axis_name="s")


@functools.partial(
    pl.kernel,
    mesh=_mesh,
    out_type=jax.ShapeDtypeStruct((NTOT, D_MODEL), jnp.float32),
    scratch_types=[
        pltpu.VMEM((B_PER_W,), jnp.int32),
        pltpu.VMEM((2, CHUNK, D_MODEL), jnp.float32),
        pltpu.SemaphoreType.DMA,
        pltpu.SemaphoreType.DMA,
        pltpu.SemaphoreType.DMA,
        pltpu.SemaphoreType.DMA,
    ],
)
def _gather_kernel(ids_hbm, table_hbm, out_hbm, idx_v, rows_v, g0, g1, o0, o1):
    wid = lax.axis_index("s") * 2 + lax.axis_index("c")
    base = wid * B_PER_W
    pltpu.sync_copy(ids_hbm.at[pl.ds(base, B_PER_W)], idx_v)

    gsem = (g0, g1)
    osem = (o0, o1)

    def gather(g, buf):
        return pltpu.make_async_copy(
            table_hbm.at[idx_v.at[pl.ds(g * CHUNK, CHUNK)]],
            rows_v.at[buf],
            gsem[buf],
        )

    def writeback(g, buf):
        return pltpu.make_async_copy(
            rows_v.at[buf],
            out_hbm.at[pl.ds(base + g * CHUNK, CHUNK)],
            osem[buf],
        )

    gather(0, 0).start()

    def round_body(r, first, last):
        g = 2 * r
        gather(g, 0).wait()
        if not first:
            writeback(g - 1, 1).wait()
        gather(g + 1, 1).start()
        writeback(g, 0).start()
        gather(g + 1, 1).wait()
        writeback(g, 0).wait()
        if not last:
            gather(g + 2, 0).start()
        writeback(g + 1, 1).start()
        return 0

    round_body(0, True, False)
    lax.fori_loop(1, NROUND - 1, lambda r, c: round_body(r, False, False), 0)
    round_body(NROUND - 1, False, True)
    writeback(NCHUNK - 1, 1).wait()


def kernel(position_ids, table):
    ids_flat = position_ids.reshape(-1).astype(jnp.int32)
    out = _gather_kernel(ids_flat, table)
    return out.reshape(BATCH, SEQ_LEN, D_MODEL)


# speedup vs baseline: 2.3659x; 1.0298x over previous
import functools

import jax
import jax.numpy as jnp
from jax import lax
from jax.experimental import pallas as pl
from jax.experimental.pallas import tpu as pltpu
from jax.experimental.pallas import tpu_sc as plsc

MAX_CONTEXT_LENGTH = 8192
D_MODEL = 1024
BATCH = 4
SEQ_LEN = 8192

NTOT = BATCH * SEQ_LEN
NW = 32
B_PER_W = NTOT // NW
CHUNK = 16
NBUF = 4
NCHUNK = B_PER_W // CHUNK
NROUND = NCHUNK // NBUF

_mesh = plsc.VectorSubcoreMesh(core_axis_name="c", subcore_axis_name="s")


@functools.partial(
    pl.kernel,
    mesh=_mesh,
    out_type=jax.ShapeDtypeStruct((NTOT, D_MODEL), jnp.float32),
    scratch_types=[
        pltpu.VMEM((B_PER_W,), jnp.int32),
        pltpu.VMEM((NBUF, CHUNK, D_MODEL), jnp.float32),
    ]
    + [pltpu.SemaphoreType.DMA] * (2 * NBUF),
)
def _gather_kernel(ids_hbm, table_hbm, out_hbm, idx_v, rows_v, *sems):
    gsem, osem = sems[:NBUF], sems[NBUF:]
    wid = lax.axis_index("s") * 2 + lax.axis_index("c")
    base = wid * B_PER_W
    pltpu.sync_copy(ids_hbm.at[pl.ds(base, B_PER_W)], idx_v)

    def gather(g, buf):
        return pltpu.make_async_copy(
            table_hbm.at[idx_v.at[pl.ds(g * CHUNK, CHUNK)]],
            rows_v.at[buf],
            gsem[buf],
        )

    def writeback(g, buf):
        return pltpu.make_async_copy(
            rows_v.at[buf],
            out_hbm.at[pl.ds(base + g * CHUNK, CHUNK)],
            osem[buf],
        )

    gather(0, 0).start()
    gather(1, 1).start()

    def step(g, j, wait_wb, start_g):
        gather(g, j).wait()
        if wait_wb:
            writeback(g - 2, (j + 2) % NBUF).wait()
        if start_g:
            gather(g + 2, (j + 2) % NBUF).start()
        writeback(g, j).start()

    for j in range(NBUF):
        step(j, j, j >= 2, True)

    def round_body(r, c):
        g0 = NBUF * r
        for j in range(NBUF):
            step(g0 + j, j, True, True)
        return c

    lax.fori_loop(1, NROUND - 1, round_body, 0)

    g0 = NBUF * (NROUND - 1)
    for j in range(NBUF):
        step(g0 + j, j, True, j < 2)
    writeback(NCHUNK - 2, 2).wait()
    writeback(NCHUNK - 1, 3).wait()


def kernel(position_ids, table):
    ids_flat = position_ids.reshape(-1).astype(jnp.int32)
    out = _gather_kernel(ids_flat, table)
    return out.reshape(BATCH, SEQ_LEN, D_MODEL)
